# baseline (speedup 1.0000x reference)
import functools

import jax
import jax.numpy as jnp
from jax import lax
from jax.experimental import pallas as pl
from jax.experimental.pallas import tpu as pltpu
from jax.experimental.pallas import tpu_sc as plsc


def _tc_body(idx_ref, d_ref, w1t_ref, b1_ref, w2t_ref, b2_ref, out_ref):
    R, K = idx_ref.shape
    HI = lax.Precision.HIGHEST
    LO = lax.Precision.DEFAULT
    kk = lax.broadcasted_iota(jnp.int32, (K, K * K), 0)
    ll = lax.broadcasted_iota(jnp.int32, (K, K * K), 1)
    picki = (ll // K == kk).astype(jnp.float32)
    pickj = (ll % K == kk).astype(jnp.float32)
    dmat = picki - pickj
    l1 = lax.broadcasted_iota(jnp.int32, (1, K * K), 1)
    ltm = (l1 % K < l1 // K).astype(jnp.float32)
    s0 = lax.broadcasted_iota(jnp.int32, (K * K, K), 0)
    s1 = lax.broadcasted_iota(jnp.int32, (K * K, K), 1)
    sred = (s0 // K == s1).astype(jnp.float32)
    r0 = lax.broadcasted_iota(jnp.int32, (K, K), 0)
    r1 = lax.broadcasted_iota(jnp.int32, (K, K), 1)
    tri = (r0 <= r1).astype(jnp.float32)
    idx = idx_ref[...]
    d = d_ref[...]
    idxf = idx.astype(jnp.float32)
    diff = jnp.dot(idxf, dmat, precision=HI)
    eq2 = (diff == 0.0).astype(jnp.float32)
    seen = jnp.dot(eq2 * ltm, sred, precision=LO)
    is_new = ((idx != 0) & (seen == 0.0)).astype(jnp.float32)
    counts = jnp.dot(is_new, tri, precision=LO)
    feat = jnp.concatenate([d, counts], axis=-1)
    h = jnp.tanh(
        lax.dot_general(
            feat, w1t_ref[...], (((1,), (1,)), ((), ())), precision=HI)
        + b1_ref[...]
    )
    logit = jnp.sum(h * w2t_ref[...], axis=-1, keepdims=True) + b2_ref[...]
    tempe = jax.nn.sigmoid(logit)
    x = -d * tempe
    x = x - jnp.max(x, axis=-1, keepdims=True)
    e = jnp.exp(x)
    p = e / jnp.sum(e, axis=-1, keepdims=True)
    p2 = jnp.dot(p, pickj, precision=HI)
    comb = jnp.dot(eq2 * p2, sred, precision=HI)
    out_ref[...] = comb


def _tc_combine(idx, d, W1, b1, W2, b2):
    R, K = idx.shape
    return pl.pallas_call(
        _tc_body,
        out_shape=jax.ShapeDtypeStruct((R, K), jnp.float32),
    )(idx, d, W1.T, b1.reshape(1, -1), W2.reshape(1, -1), b2.reshape(1, 1))


@functools.cache
def _make_sc_scatter(B, S, K, V):
    NC, NS = 2, 16
    NW = NC * NS
    assert B == NW and K % 16 == 0
    CW = 6144
    NCHUNK = V // CW
    assert NCHUNK % 2 == 0
    TW = V - NCHUNK * CW
    mesh = plsc.VectorSubcoreMesh(core_axis_name="c", subcore_axis_name="s")

    def _scatter_halves(buf, idx_v, val_v, base, width, vals_are_zero):
        for s in range(S):
            srow = jnp.full((16,), s, jnp.int32)
            for h in range(K // 16):
                iv = idx_v[s, pl.ds(h * 16, 16)]
                m = (iv >= base) & (iv < base + width)
                loc = jnp.where(m, iv - base, 0)
                if vals_are_zero:
                    vv = jnp.zeros((16,), jnp.float32)
                else:
                    vv = val_v[s, pl.ds(h * 16, 16)]
                plsc.store_scatter(buf, [srow, loc], vv, mask=m)

    @functools.partial(
        pl.kernel,
        mesh=mesh,
        out_type=jax.ShapeDtypeStruct((B, S, V), jnp.float32),
        compiler_params=pltpu.CompilerParams(needs_layout_passes=False),
        scratch_types=[
            pltpu.VMEM((S, CW), jnp.float32),
            pltpu.VMEM((S, CW), jnp.float32),
            pltpu.VMEM((S, TW), jnp.float32),
            pltpu.VMEM((S, K), jnp.int32),
            pltpu.VMEM((S, K), jnp.float32),
            pltpu.SemaphoreType.DMA,
            pltpu.SemaphoreType.DMA,
            pltpu.SemaphoreType.DMA,
        ],
    )
    def sc_scatter(zeros_hbm, idx_hbm, val_hbm, out_hbm,
                   buf_a, buf_b, tailbuf, idx_v, val_v, sem_a, sem_b, sem_p):
        b = lax.axis_index("s") * NC + lax.axis_index("c")
        cp_a = pltpu.async_copy(zeros_hbm.at[:, pl.ds(0, CW)], buf_a, sem_a)
        cp_b = pltpu.async_copy(zeros_hbm.at[:, pl.ds(0, CW)], buf_b, sem_b)
        cp_t = pltpu.async_copy(zeros_hbm.at[:, pl.ds(CW, TW)], tailbuf, sem_p)
        cp_i = pltpu.async_copy(idx_hbm.at[pl.ds(b * S, S)], idx_v, sem_p)
        cp_v = pltpu.async_copy(val_hbm.at[pl.ds(b * S, S)], val_v, sem_p)

        def _fire(buf, base, sem):
            return pltpu.async_copy(
                buf, out_hbm.at[b, :, pl.ds(base, CW)], sem)

        cp_t.wait()
        cp_i.wait()
        cp_v.wait()
        cp_a.wait()
        _scatter_halves(buf_a, idx_v, val_v, 0, CW, False)
        _fire(buf_a, 0, sem_a)
        cp_b.wait()
        _scatter_halves(buf_b, idx_v, val_v, CW, CW, False)
        _fire(buf_b, CW, sem_b)

        @pl.loop(1, NCHUNK // 2)
        def _chunk_pair(i):
            for buf, sem, par in ((buf_a, sem_a, 0), (buf_b, sem_b, 1)):
                base = (2 * i + par) * CW
                pltpu.make_async_copy(
                    buf, out_hbm.at[b, :, pl.ds(base - 2 * CW, CW)], sem
                ).wait()
                _scatter_halves(buf, idx_v, val_v, base - 2 * CW, CW, True)
                _scatter_halves(buf, idx_v, val_v, base, CW, False)
                _fire(buf, base, sem)

        base = NCHUNK * CW
        _scatter_halves(tailbuf, idx_v, val_v, base, TW, False)
        tail_cp = pltpu.async_copy(
            tailbuf, out_hbm.at[b, :, pl.ds(base, TW)], sem_p)
        pltpu.make_async_copy(
            buf_a, out_hbm.at[b, :, pl.ds(0, CW)], sem_a).wait()
        pltpu.make_async_copy(
            buf_b, out_hbm.at[b, :, pl.ds(0, CW)], sem_b).wait()
        tail_cp.wait()

    return sc_scatter


def kernel(tgt_index, knn_dists, nmt_prob, W1, b1, W2, b2):
    B, S, K = knn_dists.shape
    V = nmt_prob.shape[-1]
    R = B * S
    idx = tgt_index.reshape(R, K).astype(jnp.int32)
    d = knn_dists.reshape(R, K).astype(jnp.float32)
    vals = _tc_combine(idx, d, W1, b1, W2, b2)
    CW = 6144
    TW = V - (V // CW) * CW
    zeros_src = jnp.zeros((S, CW + TW), jnp.float32)
    return _make_sc_scatter(B, S, K, V)(zeros_src, idx, vals)


# speedup vs baseline: 1.0913x; 1.0913x over previous
import functools

import jax
import jax.numpy as jnp
from jax import lax
from jax.experimental import pallas as pl
from jax.experimental.pallas import tpu as pltpu
from jax.experimental.pallas import tpu_sc as plsc


def _tc_body(idx_ref, d_ref, w1t_ref, b1_ref, w2t_ref, b2_ref, out_ref):
    R, K = idx_ref.shape
    HI = lax.Precision.HIGHEST
    LO = lax.Precision.DEFAULT
    kk = lax.broadcasted_iota(jnp.int32, (K, K * K), 0)
    ll = lax.broadcasted_iota(jnp.int32, (K, K * K), 1)
    picki = (ll // K == kk).astype(jnp.float32)
    pickj = (ll % K == kk).astype(jnp.float32)
    dmat = picki - pickj
    l1 = lax.broadcasted_iota(jnp.int32, (1, K * K), 1)
    ltm = (l1 % K < l1 // K).astype(jnp.float32)
    s0 = lax.broadcasted_iota(jnp.int32, (K * K, K), 0)
    s1 = lax.broadcasted_iota(jnp.int32, (K * K, K), 1)
    sred = (s0 // K == s1).astype(jnp.float32)
    r0 = lax.broadcasted_iota(jnp.int32, (K, K), 0)
    r1 = lax.broadcasted_iota(jnp.int32, (K, K), 1)
    tri = (r0 <= r1).astype(jnp.float32)
    idx = idx_ref[...]
    d = d_ref[...]
    idxf = idx.astype(jnp.float32)
    diff = jnp.dot(idxf, dmat, precision=HI)
    eq2 = (diff == 0.0).astype(jnp.float32)
    seen = jnp.dot(eq2 * ltm, sred, precision=LO)
    is_new = ((idx != 0) & (seen == 0.0)).astype(jnp.float32)
    counts = jnp.dot(is_new, tri, precision=LO)
    feat = jnp.concatenate([d, counts], axis=-1)
    h = jnp.tanh(
        lax.dot_general(
            feat, w1t_ref[...], (((1,), (1,)), ((), ())), precision=HI)
        + b1_ref[...]
    )
    logit = jnp.sum(h * w2t_ref[...], axis=-1, keepdims=True) + b2_ref[...]
    tempe = jax.nn.sigmoid(logit)
    x = -d * tempe
    x = x - jnp.max(x, axis=-1, keepdims=True)
    e = jnp.exp(x)
    p = e / jnp.sum(e, axis=-1, keepdims=True)
    p2 = jnp.dot(p, pickj, precision=HI)
    comb = jnp.dot(eq2 * p2, sred, precision=HI)
    out_ref[...] = comb


def _tc_combine(idx, d, W1, b1, W2, b2):
    R, K = idx.shape
    return pl.pallas_call(
        _tc_body,
        out_shape=jax.ShapeDtypeStruct((R, K), jnp.float32),
    )(idx, d, W1.T, b1.reshape(1, -1), W2.reshape(1, -1), b2.reshape(1, 1))


@functools.cache
def _make_sc_scatter(B, S, K, V):
    NC, NS = 2, 16
    NW = NC * NS
    assert B == NW and K % 16 == 0
    CW = 6144
    NCHUNK = V // CW
    assert NCHUNK % 2 == 0
    TW = V - NCHUNK * CW
    mesh = plsc.VectorSubcoreMesh(core_axis_name="c", subcore_axis_name="s")

    def _scatter_halves(buf, idx_v, val_v, base, width, vals_are_zero):
        for s in range(S):
            srow = jnp.full((16,), s, jnp.int32)
            for h in range(K // 16):
                iv = idx_v[s, pl.ds(h * 16, 16)]
                m = (iv >= base) & (iv < base + width)
                loc = jnp.where(m, iv - base, 0)
                if vals_are_zero:
                    vv = jnp.zeros((16,), jnp.float32)
                else:
                    vv = val_v[s, pl.ds(h * 16, 16)]
                plsc.store_scatter(buf, [srow, loc], vv, mask=m)

    @functools.partial(
        pl.kernel,
        mesh=mesh,
        out_type=jax.ShapeDtypeStruct((B, S, V), jnp.float32),
        compiler_params=pltpu.CompilerParams(needs_layout_passes=False),
        scratch_types=[
            pltpu.VMEM((S, CW), jnp.float32),
            pltpu.VMEM((S, CW), jnp.float32),
            pltpu.VMEM((S, TW), jnp.float32),
            pltpu.VMEM((S, K), jnp.int32),
            pltpu.VMEM((S, K), jnp.float32),
            pltpu.SemaphoreType.DMA,
            pltpu.SemaphoreType.DMA,
            pltpu.SemaphoreType.DMA,
        ],
    )
    def sc_scatter(zeros_hbm, idx_hbm, val_hbm, out_hbm,
                   buf_a, buf_b, tailbuf, idx_v, val_v, sem_a, sem_b, sem_p):
        b = lax.axis_index("s") * NC + lax.axis_index("c")
        pre = [
            pltpu.async_copy(zeros_hbm.at[:, pl.ds(0, CW)], buf_a, sem_p),
            pltpu.async_copy(zeros_hbm.at[:, pl.ds(0, CW)], buf_b, sem_p),
            pltpu.async_copy(zeros_hbm.at[:, pl.ds(CW, TW)], tailbuf, sem_p),
            pltpu.async_copy(idx_hbm.at[pl.ds(b * S, S)], idx_v, sem_p),
            pltpu.async_copy(val_hbm.at[pl.ds(b * S, S)], val_v, sem_p),
        ]
        for cp in pre:
            cp.wait()

        def _fire(buf, base, sem):
            return pltpu.async_copy(
                buf, out_hbm.at[b, :, pl.ds(base, CW)], sem)

        _scatter_halves(buf_a, idx_v, val_v, 0, CW, False)
        _fire(buf_a, 0, sem_a)
        _scatter_halves(buf_b, idx_v, val_v, CW, CW, False)
        _fire(buf_b, CW, sem_b)

        @pl.loop(1, NCHUNK // 2)
        def _chunk_pair(i):
            for buf, sem, par in ((buf_a, sem_a, 0), (buf_b, sem_b, 1)):
                base = (2 * i + par) * CW
                pltpu.make_async_copy(
                    buf, out_hbm.at[b, :, pl.ds(base - 2 * CW, CW)], sem
                ).wait()
                _scatter_halves(buf, idx_v, val_v, base - 2 * CW, CW, True)
                _scatter_halves(buf, idx_v, val_v, base, CW, False)
                _fire(buf, base, sem)

        base = NCHUNK * CW
        _scatter_halves(tailbuf, idx_v, val_v, base, TW, False)
        tail_cp = pltpu.async_copy(
            tailbuf, out_hbm.at[b, :, pl.ds(base, TW)], sem_p)
        pltpu.make_async_copy(
            buf_a, out_hbm.at[b, :, pl.ds(0, CW)], sem_a).wait()
        pltpu.make_async_copy(
            buf_b, out_hbm.at[b, :, pl.ds(0, CW)], sem_b).wait()
        tail_cp.wait()

    return sc_scatter


def kernel(tgt_index, knn_dists, nmt_prob, W1, b1, W2, b2):
    B, S, K = knn_dists.shape
    V = nmt_prob.shape[-1]
    R = B * S
    idx = tgt_index.reshape(R, K).astype(jnp.int32)
    d = knn_dists.reshape(R, K).astype(jnp.float32)
    vals = _tc_combine(idx, d, W1, b1, W2, b2)
    CW = 6144
    TW = V - (V // CW) * CW
    zeros_src = jnp.zeros((S, CW + TW), jnp.float32)
    return _make_sc_scatter(B, S, K, V)(zeros_src, idx, vals)


# speedup vs baseline: 1.1445x; 1.0487x over previous
import functools

import jax
import jax.numpy as jnp
from jax import lax
from jax.experimental import pallas as pl
from jax.experimental.pallas import tpu as pltpu
from jax.experimental.pallas import tpu_sc as plsc


def _tc_body(idx_ref, d_ref, w1t_ref, b1_ref, w2t_ref, b2_ref, out_ref):
    R, K = idx_ref.shape
    HI = lax.Precision.HIGHEST
    LO = lax.Precision.DEFAULT
    kk = lax.broadcasted_iota(jnp.int32, (K, K * K), 0)
    ll = lax.broadcasted_iota(jnp.int32, (K, K * K), 1)
    picki = (ll // K == kk).astype(jnp.float32)
    pickj = (ll % K == kk).astype(jnp.float32)
    dmat = picki - pickj
    l1 = lax.broadcasted_iota(jnp.int32, (1, K * K), 1)
    ltm = (l1 % K < l1 // K).astype(jnp.float32)
    s0 = lax.broadcasted_iota(jnp.int32, (K * K, K), 0)
    s1 = lax.broadcasted_iota(jnp.int32, (K * K, K), 1)
    sred = (s0 // K == s1).astype(jnp.float32)
    r0 = lax.broadcasted_iota(jnp.int32, (K, K), 0)
    r1 = lax.broadcasted_iota(jnp.int32, (K, K), 1)
    tri = (r0 <= r1).astype(jnp.float32)
    idx = idx_ref[...]
    d = d_ref[...]
    idxf = idx.astype(jnp.float32)
    diff = jnp.dot(idxf, dmat, precision=HI)
    eq2 = (diff == 0.0).astype(jnp.float32)
    seen = jnp.dot(eq2 * ltm, sred, precision=LO)
    is_new = ((idx != 0) & (seen == 0.0)).astype(jnp.float32)
    counts = jnp.dot(is_new, tri, precision=LO)
    feat = jnp.concatenate([d, counts], axis=-1)
    h = jnp.tanh(
        lax.dot_general(
            feat, w1t_ref[...], (((1,), (1,)), ((), ())), precision=HI)
        + b1_ref[...]
    )
    logit = jnp.sum(h * w2t_ref[...], axis=-1, keepdims=True) + b2_ref[...]
    tempe = jax.nn.sigmoid(logit)
    x = -d * tempe
    x = x - jnp.max(x, axis=-1, keepdims=True)
    e = jnp.exp(x)
    p = e / jnp.sum(e, axis=-1, keepdims=True)
    p2 = jnp.dot(p, pickj, precision=HI)
    comb = jnp.dot(eq2 * p2, sred, precision=HI)
    out_ref[...] = comb


def _tc_combine(idx, d, W1, b1, W2, b2):
    R, K = idx.shape
    return pl.pallas_call(
        _tc_body,
        out_shape=jax.ShapeDtypeStruct((R, K), jnp.float32),
    )(idx, d, W1.T, b1.reshape(1, -1), W2.reshape(1, -1), b2.reshape(1, 1))


@functools.cache
def _make_sc_scatter(B, S, K, V):
    NC, NS = 2, 16
    NW = NC * NS
    assert B == NW and K % 16 == 0
    CW = 4096
    NCHUNK = V // CW
    assert NCHUNK % 2 == 0
    TW = V - NCHUNK * CW
    mesh = plsc.VectorSubcoreMesh(core_axis_name="c", subcore_axis_name="s")

    def _scatter_halves(buf, idx_v, val_v, base, width, vals_are_zero):
        for s in range(S):
            srow = jnp.full((16,), s, jnp.int32)
            for h in range(K // 16):
                iv = idx_v[s, pl.ds(h * 16, 16)]
                m = (iv >= base) & (iv < base + width)
                loc = jnp.where(m, iv - base, 0)
                if vals_are_zero:
                    vv = jnp.zeros((16,), jnp.float32)
                else:
                    vv = val_v[s, pl.ds(h * 16, 16)]
                plsc.store_scatter(buf, [srow, loc], vv, mask=m)

    @functools.partial(
        pl.kernel,
        mesh=mesh,
        out_type=jax.ShapeDtypeStruct((B, S, V), jnp.float32),
        compiler_params=pltpu.CompilerParams(needs_layout_passes=False),
        scratch_types=[
            pltpu.VMEM((S, CW), jnp.float32),
            pltpu.VMEM((S, CW), jnp.float32),
            pltpu.VMEM((S, TW), jnp.float32),
            pltpu.VMEM((S, K), jnp.int32),
            pltpu.VMEM((S, K), jnp.float32),
            pltpu.SemaphoreType.DMA,
            pltpu.SemaphoreType.DMA,
            pltpu.SemaphoreType.DMA,
        ],
    )
    def sc_scatter(zeros_hbm, idx_hbm, val_hbm, out_hbm,
                   buf_a, buf_b, tailbuf, idx_v, val_v, sem_a, sem_b, sem_p):
        b = lax.axis_index("s") * NC + lax.axis_index("c")
        pre = [
            pltpu.async_copy(zeros_hbm.at[:, pl.ds(0, CW)], buf_a, sem_p),
            pltpu.async_copy(zeros_hbm.at[:, pl.ds(0, CW)], buf_b, sem_p),
            pltpu.async_copy(zeros_hbm.at[:, pl.ds(CW, TW)], tailbuf, sem_p),
            pltpu.async_copy(idx_hbm.at[pl.ds(b * S, S)], idx_v, sem_p),
            pltpu.async_copy(val_hbm.at[pl.ds(b * S, S)], val_v, sem_p),
        ]
        for cp in pre:
            cp.wait()

        def _fire(buf, base, sem):
            return pltpu.async_copy(
                buf, out_hbm.at[b, :, pl.ds(base, CW)], sem)

        _scatter_halves(buf_a, idx_v, val_v, 0, CW, False)
        _fire(buf_a, 0, sem_a)
        _scatter_halves(buf_b, idx_v, val_v, CW, CW, False)
        _fire(buf_b, CW, sem_b)

        @pl.loop(1, NCHUNK // 2)
        def _chunk_pair(i):
            for buf, sem, par in ((buf_a, sem_a, 0), (buf_b, sem_b, 1)):
                base = (2 * i + par) * CW
                pltpu.make_async_copy(
                    buf, out_hbm.at[b, :, pl.ds(base - 2 * CW, CW)], sem
                ).wait()
                _scatter_halves(buf, idx_v, val_v, base - 2 * CW, CW, True)
                _scatter_halves(buf, idx_v, val_v, base, CW, False)
                _fire(buf, base, sem)

        base = NCHUNK * CW
        _scatter_halves(tailbuf, idx_v, val_v, base, TW, False)
        tail_cp = pltpu.async_copy(
            tailbuf, out_hbm.at[b, :, pl.ds(base, TW)], sem_p)
        pltpu.make_async_copy(
            buf_a, out_hbm.at[b, :, pl.ds(0, CW)], sem_a).wait()
        pltpu.make_async_copy(
            buf_b, out_hbm.at[b, :, pl.ds(0, CW)], sem_b).wait()
        tail_cp.wait()

    return sc_scatter


def kernel(tgt_index, knn_dists, nmt_prob, W1, b1, W2, b2):
    B, S, K = knn_dists.shape
    V = nmt_prob.shape[-1]
    R = B * S
    idx = tgt_index.reshape(R, K).astype(jnp.int32)
    d = knn_dists.reshape(R, K).astype(jnp.float32)
    vals = _tc_combine(idx, d, W1, b1, W2, b2)
    CW = 4096
    TW = V - (V // CW) * CW
    zeros_src = jnp.zeros((S, CW + TW), jnp.float32)
    return _make_sc_scatter(B, S, K, V)(zeros_src, idx, vals)


# speedup vs baseline: 1.2025x; 1.0507x over previous
import functools

import jax
import jax.numpy as jnp
from jax import lax
from jax.experimental import pallas as pl
from jax.experimental.pallas import tpu as pltpu
from jax.experimental.pallas import tpu_sc as plsc


def _tc_body(idx_ref, d_ref, w1t_ref, b1_ref, w2t_ref, b2_ref, out_ref):
    R, K = idx_ref.shape
    HI = lax.Precision.HIGHEST
    LO = lax.Precision.DEFAULT
    kk = lax.broadcasted_iota(jnp.int32, (K, K * K), 0)
    ll = lax.broadcasted_iota(jnp.int32, (K, K * K), 1)
    picki = (ll // K == kk).astype(jnp.float32)
    pickj = (ll % K == kk).astype(jnp.float32)
    dmat = picki - pickj
    l1 = lax.broadcasted_iota(jnp.int32, (1, K * K), 1)
    ltm = (l1 % K < l1 // K).astype(jnp.float32)
    s0 = lax.broadcasted_iota(jnp.int32, (K * K, K), 0)
    s1 = lax.broadcasted_iota(jnp.int32, (K * K, K), 1)
    sred = (s0 // K == s1).astype(jnp.float32)
    r0 = lax.broadcasted_iota(jnp.int32, (K, K), 0)
    r1 = lax.broadcasted_iota(jnp.int32, (K, K), 1)
    tri = (r0 <= r1).astype(jnp.float32)
    idx = idx_ref[...]
    d = d_ref[...]
    idxf = idx.astype(jnp.float32)
    diff = jnp.dot(idxf, dmat, precision=HI)
    eq2 = (diff == 0.0).astype(jnp.float32)
    seen = jnp.dot(eq2 * ltm, sred, precision=LO)
    is_new = ((idx != 0) & (seen == 0.0)).astype(jnp.float32)
    counts = jnp.dot(is_new, tri, precision=LO)
    feat = jnp.concatenate([d, counts], axis=-1)
    h = jnp.tanh(
        lax.dot_general(
            feat, w1t_ref[...], (((1,), (1,)), ((), ())), precision=HI)
        + b1_ref[...]
    )
    logit = jnp.sum(h * w2t_ref[...], axis=-1, keepdims=True) + b2_ref[...]
    tempe = jax.nn.sigmoid(logit)
    x = -d * tempe
    x = x - jnp.max(x, axis=-1, keepdims=True)
    e = jnp.exp(x)
    p = e / jnp.sum(e, axis=-1, keepdims=True)
    p2 = jnp.dot(p, pickj, precision=HI)
    comb = jnp.dot(eq2 * p2, sred, precision=HI)
    out_ref[...] = comb


def _tc_combine(idx, d, W1, b1, W2, b2):
    R, K = idx.shape
    return pl.pallas_call(
        _tc_body,
        out_shape=jax.ShapeDtypeStruct((R, K), jnp.float32),
    )(idx, d, W1.T, b1.reshape(1, -1), W2.reshape(1, -1), b2.reshape(1, 1))


@functools.cache
def _make_sc_scatter(B, S, K, V):
    NC, NS = 2, 16
    NW = NC * NS
    assert B == NW and K % 16 == 0
    CW = 2048
    NCHUNK = V // CW
    assert NCHUNK % 2 == 0
    TW = V - NCHUNK * CW
    mesh = plsc.VectorSubcoreMesh(core_axis_name="c", subcore_axis_name="s")

    def _scatter_halves(buf, idx_v, val_v, base, width, vals_are_zero):
        for s in range(S):
            srow = jnp.full((16,), s, jnp.int32)
            for h in range(K // 16):
                iv = idx_v[s, pl.ds(h * 16, 16)]
                m = (iv >= base) & (iv < base + width)
                loc = jnp.where(m, iv - base, 0)
                if vals_are_zero:
                    vv = jnp.zeros((16,), jnp.float32)
                else:
                    vv = val_v[s, pl.ds(h * 16, 16)]
                plsc.store_scatter(buf, [srow, loc], vv, mask=m)

    @functools.partial(
        pl.kernel,
        mesh=mesh,
        out_type=jax.ShapeDtypeStruct((B, S, V), jnp.float32),
        compiler_params=pltpu.CompilerParams(needs_layout_passes=False),
        scratch_types=[
            pltpu.VMEM((S, CW), jnp.float32),
            pltpu.VMEM((S, CW), jnp.float32),
            pltpu.VMEM((S, TW), jnp.float32),
            pltpu.VMEM((S, K), jnp.int32),
            pltpu.VMEM((S, K), jnp.float32),
            pltpu.SemaphoreType.DMA,
            pltpu.SemaphoreType.DMA,
            pltpu.SemaphoreType.DMA,
        ],
    )
    def sc_scatter(zeros_hbm, idx_hbm, val_hbm, out_hbm,
                   buf_a, buf_b, tailbuf, idx_v, val_v, sem_a, sem_b, sem_p):
        b = lax.axis_index("s") * NC + lax.axis_index("c")
        pre = [
            pltpu.async_copy(zeros_hbm.at[:, pl.ds(0, CW)], buf_a, sem_p),
            pltpu.async_copy(zeros_hbm.at[:, pl.ds(0, CW)], buf_b, sem_p),
            pltpu.async_copy(zeros_hbm.at[:, pl.ds(CW, TW)], tailbuf, sem_p),
            pltpu.async_copy(idx_hbm.at[pl.ds(b * S, S)], idx_v, sem_p),
            pltpu.async_copy(val_hbm.at[pl.ds(b * S, S)], val_v, sem_p),
        ]
        for cp in pre:
            cp.wait()

        def _fire(buf, base, sem):
            return pltpu.async_copy(
                buf, out_hbm.at[b, :, pl.ds(base, CW)], sem)

        _scatter_halves(buf_a, idx_v, val_v, 0, CW, False)
        _fire(buf_a, 0, sem_a)
        _scatter_halves(buf_b, idx_v, val_v, CW, CW, False)
        _fire(buf_b, CW, sem_b)

        @pl.loop(1, NCHUNK // 2)
        def _chunk_pair(i):
            for buf, sem, par in ((buf_a, sem_a, 0), (buf_b, sem_b, 1)):
                base = (2 * i + par) * CW
                pltpu.make_async_copy(
                    buf, out_hbm.at[b, :, pl.ds(base - 2 * CW, CW)], sem
                ).wait()
                _scatter_halves(buf, idx_v, val_v, base - 2 * CW, CW, True)
                _scatter_halves(buf, idx_v, val_v, base, CW, False)
                _fire(buf, base, sem)

        base = NCHUNK * CW
        _scatter_halves(tailbuf, idx_v, val_v, base, TW, False)
        tail_cp = pltpu.async_copy(
            tailbuf, out_hbm.at[b, :, pl.ds(base, TW)], sem_p)
        pltpu.make_async_copy(
            buf_a, out_hbm.at[b, :, pl.ds(0, CW)], sem_a).wait()
        pltpu.make_async_copy(
            buf_b, out_hbm.at[b, :, pl.ds(0, CW)], sem_b).wait()
        tail_cp.wait()

    return sc_scatter


def kernel(tgt_index, knn_dists, nmt_prob, W1, b1, W2, b2):
    B, S, K = knn_dists.shape
    V = nmt_prob.shape[-1]
    R = B * S
    idx = tgt_index.reshape(R, K).astype(jnp.int32)
    d = knn_dists.reshape(R, K).astype(jnp.float32)
    vals = _tc_combine(idx, d, W1, b1, W2, b2)
    CW = 2048
    TW = V - (V // CW) * CW
    zeros_src = jnp.zeros((S, CW + TW), jnp.float32)
    return _make_sc_scatter(B, S, K, V)(zeros_src, idx, vals)
